# baseline (speedup 1.0000x reference)
import functools

import jax
import jax.numpy as jnp
from jax import lax
from jax.experimental import pallas as pl
from jax.experimental.pallas import tpu as pltpu
from jax.experimental.pallas import tpu_sc as plsc

FROM_DIM = 64
TO_DIM = 64
BATCH = 16384
FIELDS = 26
VOCAB = 1000000

NC, NS = 2, 16
NW = NC * NS
BW = BATCH // NW
BB = 16
NCH = BW // BB

T_BLK = 512
T_GRID = -(-VOCAB // T_BLK)
P3_BM = 256


def _transform_body(wt_ref, b_ref, x_ref, o_ref):
    y = jnp.dot(wt_ref[...], x_ref[...], preferred_element_type=jnp.float32)
    y = y + b_ref[...]
    yt = y.T
    o_ref[...] = jnp.concatenate([yt, yt], axis=1)


def _tc_transform(tableT, W, b):
    return pl.pallas_call(
        _transform_body,
        grid=(T_GRID,),
        in_specs=[
            pl.BlockSpec((TO_DIM, FROM_DIM), lambda i: (0, 0)),
            pl.BlockSpec((TO_DIM, 1), lambda i: (0, 0)),
            pl.BlockSpec((FROM_DIM, T_BLK), lambda i: (0, i)),
        ],
        out_specs=pl.BlockSpec((T_BLK, 2 * TO_DIM), lambda i: (i, 0)),
        out_shape=jax.ShapeDtypeStruct((VOCAB, 2 * TO_DIM), jnp.float32),
    )(W, b, tableT)


def _sc_gather(t2, idx2d):
    mesh = plsc.VectorSubcoreMesh(
        core_axis_name="c", subcore_axis_name="s",
        num_cores=NC, num_subcores=NS)

    @functools.partial(
        pl.kernel,
        out_type=jax.ShapeDtypeStruct((BATCH, FIELDS, 2 * TO_DIM), jnp.float32),
        mesh=mesh,
        scratch_types=[
            pltpu.VMEM((BB, FIELDS), jnp.int32),
            pltpu.VMEM((BB, FIELDS, 2 * TO_DIM), jnp.float32),
            pltpu.SemaphoreType.DMA,
        ],
    )
    def gather_kernel(t2_hbm, idx_hbm, emb_hbm, idx_v, rows_v, sem):
        wid = lax.axis_index("s") * NC + lax.axis_index("c")
        b0 = wid * BW

        def chunk(g, carry):
            bb0 = b0 + g * BB
            pltpu.sync_copy(idx_hbm.at[pl.ds(bb0, BB)], idx_v)
            descs = [
                pltpu.async_copy(
                    t2_hbm.at[idx_v.at[bb]], rows_v.at[bb], sem)
                for bb in range(BB)
            ]
            for d in descs:
                d.wait()
            pltpu.sync_copy(rows_v, emb_hbm.at[pl.ds(bb0, BB)])
            return carry

        lax.fori_loop(0, NCH, chunk, 0)

    return gather_kernel(t2, idx2d)


def _p3_body(x_ref, o_ref):
    x = x_ref[...]
    for f in range(FIELDS):
        o_ref[f] = x[:, f, :TO_DIM].T


def _tc_finalize(emb2):
    return pl.pallas_call(
        _p3_body,
        grid=(BATCH // P3_BM,),
        in_specs=[
            pl.BlockSpec((P3_BM, FIELDS, 2 * TO_DIM), lambda i: (i, 0, 0)),
        ],
        out_specs=pl.BlockSpec((FIELDS, TO_DIM, P3_BM), lambda i: (0, 0, i)),
        out_shape=jax.ShapeDtypeStruct((FIELDS, TO_DIM, BATCH), jnp.float32),
    )(emb2)


def kernel(indexes, table, W, b):
    t2 = _tc_transform(table.T, W, b.reshape(TO_DIM, 1))
    emb2 = _sc_gather(t2, indexes.astype(jnp.int32))
    out_t = _tc_finalize(emb2)
    return jnp.transpose(out_t, (2, 0, 1))


# speedup vs baseline: 2.1393x; 2.1393x over previous
import functools

import jax
import jax.numpy as jnp
from jax import lax
from jax.experimental import pallas as pl
from jax.experimental.pallas import tpu as pltpu
from jax.experimental.pallas import tpu_sc as plsc

FROM_DIM = 64
TO_DIM = 64
BATCH = 16384
FIELDS = 26
VOCAB = 1000000

NC, NS = 2, 16
NW = NC * NS
BW = BATCH // NW
BB = 16
NCH = BW // BB

T_BLK = 2048
SPLIT = 244 * T_BLK
T2_ROWS = VOCAB - SPLIT
T_GRID = -(-T2_ROWS // T_BLK)
P3_BM = 256


def _transform_body(wt_ref, b_ref, x1_ref, x2_ref, o_ref):
    dn = (((0,), (0,)), ((), ()))
    y1 = lax.dot_general(x1_ref[...], wt_ref[...], dn,
                         preferred_element_type=jnp.float32) + b_ref[...]
    y2 = lax.dot_general(x2_ref[...], wt_ref[...], dn,
                         preferred_element_type=jnp.float32) + b_ref[...]
    o_ref[...] = jnp.concatenate([y1, y2], axis=1)


def _tc_transform(tableT, Wt, b):
    return pl.pallas_call(
        _transform_body,
        grid=(T_GRID,),
        in_specs=[
            pl.BlockSpec((FROM_DIM, TO_DIM), lambda i: (0, 0)),
            pl.BlockSpec((1, TO_DIM), lambda i: (0, 0)),
            pl.BlockSpec((FROM_DIM, T_BLK), lambda i: (0, i)),
            pl.BlockSpec((FROM_DIM, T_BLK), lambda i: (0, 244 + i)),
        ],
        out_specs=pl.BlockSpec((T_BLK, 2 * TO_DIM), lambda i: (i, 0)),
        out_shape=jax.ShapeDtypeStruct((T2_ROWS, 2 * TO_DIM), jnp.float32),
    )(Wt, b, tableT, tableT)


def _sc_gather(t2, idx2d):
    mesh = plsc.VectorSubcoreMesh(
        core_axis_name="c", subcore_axis_name="s",
        num_cores=NC, num_subcores=NS)

    @functools.partial(
        pl.kernel,
        out_type=jax.ShapeDtypeStruct((BATCH, FIELDS, 2 * TO_DIM), jnp.float32),
        mesh=mesh,
        scratch_types=[
            pltpu.VMEM((BB, FIELDS), jnp.int32),
            pltpu.VMEM((BB, FIELDS, 2 * TO_DIM), jnp.float32),
            pltpu.SemaphoreType.DMA,
        ],
    )
    def gather_kernel(t2_hbm, idx_hbm, emb_hbm, idx_v, rows_v, sem):
        wid = lax.axis_index("s") * NC + lax.axis_index("c")
        b0 = wid * BW

        def chunk(g, carry):
            bb0 = b0 + g * BB
            pltpu.sync_copy(idx_hbm.at[pl.ds(bb0, BB)], idx_v)
            descs = [
                pltpu.async_copy(
                    t2_hbm.at[idx_v.at[bb]], rows_v.at[bb], sem)
                for bb in range(BB)
            ]
            for d in descs:
                d.wait()
            pltpu.sync_copy(rows_v, emb_hbm.at[pl.ds(bb0, BB)])
            return carry

        lax.fori_loop(0, NCH, chunk, 0)

    return gather_kernel(t2, idx2d)


def _p3_body(x_ref, idx_ref, o_ref):
    x = x_ref[...]
    hi = idx_ref[...] >= SPLIT
    for f in range(FIELDS):
        xf = x[:, f, :]
        sel = jnp.where(hi[:, f][:, None], xf[:, TO_DIM:], xf[:, :TO_DIM])
        o_ref[f] = sel.T


def _tc_finalize(emb2, idx2d):
    return pl.pallas_call(
        _p3_body,
        grid=(BATCH // P3_BM,),
        in_specs=[
            pl.BlockSpec((P3_BM, FIELDS, 2 * TO_DIM), lambda i: (i, 0, 0)),
            pl.BlockSpec((P3_BM, FIELDS), lambda i: (i, 0)),
        ],
        out_specs=pl.BlockSpec((FIELDS, TO_DIM, P3_BM), lambda i: (0, 0, i)),
        out_shape=jax.ShapeDtypeStruct((FIELDS, TO_DIM, BATCH), jnp.float32),
    )(emb2, idx2d)


def kernel(indexes, table, W, b):
    idx2d = indexes.astype(jnp.int32)
    k2d = jnp.where(idx2d >= SPLIT, idx2d - SPLIT, idx2d)
    t2 = _tc_transform(table.T, W.T, b.reshape(1, TO_DIM))
    emb2 = _sc_gather(t2, k2d)
    out_t = _tc_finalize(emb2, idx2d)
    return jnp.transpose(out_t, (2, 0, 1))


# speedup vs baseline: 2.4396x; 1.1404x over previous
import functools

import jax
import jax.numpy as jnp
from jax import lax
from jax.experimental import pallas as pl
from jax.experimental.pallas import tpu as pltpu
from jax.experimental.pallas import tpu_sc as plsc

FROM_DIM = 64
TO_DIM = 64
BATCH = 16384
FIELDS = 26
VOCAB = 1000000

NC, NS = 2, 16
NW = NC * NS
BW = BATCH // NW
BB = 16
NCH = BW // BB

T_BLK = 4096
SPLIT = 122 * T_BLK
T2_ROWS = VOCAB - SPLIT
T_GRID = -(-T2_ROWS // T_BLK)
P3_BM = 512


def _transform_body(wt_ref, b_ref, x1_ref, x2_ref, o_ref):
    dn = (((0,), (0,)), ((), ()))
    y1 = lax.dot_general(x1_ref[...], wt_ref[...], dn,
                         preferred_element_type=jnp.float32) + b_ref[...]
    y2 = lax.dot_general(x2_ref[...], wt_ref[...], dn,
                         preferred_element_type=jnp.float32) + b_ref[...]
    o_ref[...] = jnp.concatenate([y1, y2], axis=1)


def _tc_transform(tableT, Wt, b):
    return pl.pallas_call(
        _transform_body,
        grid=(T_GRID,),
        in_specs=[
            pl.BlockSpec((FROM_DIM, TO_DIM), lambda i: (0, 0)),
            pl.BlockSpec((1, TO_DIM), lambda i: (0, 0)),
            pl.BlockSpec((FROM_DIM, T_BLK), lambda i: (0, i)),
            pl.BlockSpec((FROM_DIM, T_BLK), lambda i: (0, 122 + i)),
        ],
        out_specs=pl.BlockSpec((T_BLK, 2 * TO_DIM), lambda i: (i, 0)),
        out_shape=jax.ShapeDtypeStruct((T2_ROWS, 2 * TO_DIM), jnp.float32),
    )(Wt, b, tableT, tableT)


def _sc_gather(t2, idx2d):
    mesh = plsc.VectorSubcoreMesh(
        core_axis_name="c", subcore_axis_name="s",
        num_cores=NC, num_subcores=NS)

    @functools.partial(
        pl.kernel,
        out_type=jax.ShapeDtypeStruct((BATCH, FIELDS, 2 * TO_DIM), jnp.float32),
        mesh=mesh,
        scratch_types=[
            pltpu.VMEM((BB, FIELDS), jnp.int32),
            pltpu.VMEM((BB, FIELDS, 2 * TO_DIM), jnp.float32),
            pltpu.SemaphoreType.DMA,
        ],
    )
    def gather_kernel(t2_hbm, idx_hbm, emb_hbm, idx_v, rows_v, sem):
        wid = lax.axis_index("s") * NC + lax.axis_index("c")
        b0 = wid * BW

        def chunk(g, carry):
            bb0 = b0 + g * BB
            pltpu.sync_copy(idx_hbm.at[pl.ds(bb0, BB)], idx_v)
            descs = [
                pltpu.async_copy(
                    t2_hbm.at[idx_v.at[bb]], rows_v.at[bb], sem)
                for bb in range(BB)
            ]
            for d in descs:
                d.wait()
            pltpu.sync_copy(rows_v, emb_hbm.at[pl.ds(bb0, BB)])
            return carry

        lax.fori_loop(0, NCH, chunk, 0)

    return gather_kernel(t2, idx2d)


def _p3_body(x_ref, idx_ref, eye_ref, o_ref):
    x = x_ref[...]
    hi = idx_ref[...] >= SPLIT
    eye = eye_ref[...]
    for f in range(FIELDS):
        xf = x[:, f, :]
        sel = jnp.where(hi[:, f][:, None], xf[:, TO_DIM:], xf[:, :TO_DIM])
        o_ref[f] = lax.dot_general(
            eye, sel, (((1,), (1,)), ((), ())),
            preferred_element_type=jnp.float32)


def _tc_finalize(emb2, idx2d, eye):
    return pl.pallas_call(
        _p3_body,
        grid=(BATCH // P3_BM,),
        in_specs=[
            pl.BlockSpec((P3_BM, FIELDS, 2 * TO_DIM), lambda i: (i, 0, 0)),
            pl.BlockSpec((P3_BM, FIELDS), lambda i: (i, 0)),
            pl.BlockSpec((TO_DIM, TO_DIM), lambda i: (0, 0)),
        ],
        out_specs=pl.BlockSpec((FIELDS, TO_DIM, P3_BM), lambda i: (0, 0, i)),
        out_shape=jax.ShapeDtypeStruct((FIELDS, TO_DIM, BATCH), jnp.float32),
    )(emb2, idx2d, eye)


def kernel(indexes, table, W, b):
    idx2d = indexes.astype(jnp.int32)
    k2d = jnp.where(idx2d >= SPLIT, idx2d - SPLIT, idx2d)
    t2 = _tc_transform(table.T, W.T, b.reshape(1, TO_DIM))
    emb2 = _sc_gather(t2, k2d)
    out_t = _tc_finalize(emb2, idx2d, jnp.eye(TO_DIM, dtype=jnp.float32))
    return jnp.transpose(out_t, (2, 0, 1))


# speedup vs baseline: 2.4925x; 1.0217x over previous
import functools

import jax
import jax.numpy as jnp
from jax import lax
from jax.experimental import pallas as pl
from jax.experimental.pallas import tpu as pltpu
from jax.experimental.pallas import tpu_sc as plsc

FROM_DIM = 64
TO_DIM = 64
BATCH = 16384
FIELDS = 26
VOCAB = 1000000

NC, NS = 2, 16
NW = NC * NS
BW = BATCH // NW
BB = 8
NPAIR = BW // (2 * BB)

T_BLK = 4096
SPLIT = 122 * T_BLK
T2_ROWS = VOCAB - SPLIT
T_GRID = -(-T2_ROWS // T_BLK)
P3_BM = 512


def _transform_body(wt_ref, b_ref, x1_ref, x2_ref, o_ref):
    dn = (((0,), (0,)), ((), ()))
    y1 = lax.dot_general(x1_ref[...], wt_ref[...], dn,
                         preferred_element_type=jnp.float32) + b_ref[...]
    y2 = lax.dot_general(x2_ref[...], wt_ref[...], dn,
                         preferred_element_type=jnp.float32) + b_ref[...]
    o_ref[...] = jnp.concatenate([y1, y2], axis=1)


def _tc_transform(tableT, Wt, b):
    return pl.pallas_call(
        _transform_body,
        grid=(T_GRID,),
        in_specs=[
            pl.BlockSpec((FROM_DIM, TO_DIM), lambda i: (0, 0)),
            pl.BlockSpec((1, TO_DIM), lambda i: (0, 0)),
            pl.BlockSpec((FROM_DIM, T_BLK), lambda i: (0, i)),
            pl.BlockSpec((FROM_DIM, T_BLK), lambda i: (0, 122 + i)),
        ],
        out_specs=pl.BlockSpec((T_BLK, 2 * TO_DIM), lambda i: (i, 0)),
        out_shape=jax.ShapeDtypeStruct((T2_ROWS, 2 * TO_DIM), jnp.float32),
    )(Wt, b, tableT, tableT)


def _sc_gather(t2, idx2d):
    mesh = plsc.VectorSubcoreMesh(
        core_axis_name="c", subcore_axis_name="s",
        num_cores=NC, num_subcores=NS)

    @functools.partial(
        pl.kernel,
        out_type=jax.ShapeDtypeStruct((BATCH, FIELDS, 2 * TO_DIM), jnp.float32),
        mesh=mesh,
        scratch_types=[
            pltpu.VMEM((BB, FIELDS), jnp.int32),
            pltpu.VMEM((BB, FIELDS), jnp.int32),
            pltpu.VMEM((BB, FIELDS, 2 * TO_DIM), jnp.float32),
            pltpu.VMEM((BB, FIELDS, 2 * TO_DIM), jnp.float32),
            pltpu.SemaphoreType.DMA,
            pltpu.SemaphoreType.DMA,
            pltpu.SemaphoreType.DMA,
            pltpu.SemaphoreType.DMA,
        ],
    )
    def gather_kernel(t2_hbm, idx_hbm, emb_hbm,
                      idx_a, idx_b, rows_a, rows_b,
                      sem_a, sem_b, sem_wa, sem_wb):
        wid = lax.axis_index("s") * NC + lax.axis_index("c")
        b0 = wid * BW

        def pair(p, carry):
            a0 = b0 + (2 * p) * BB
            c0 = a0 + BB
            pltpu.sync_copy(idx_hbm.at[pl.ds(a0, BB)], idx_a)
            da = [
                pltpu.async_copy(t2_hbm.at[idx_a.at[bb]], rows_a.at[bb], sem_a)
                for bb in range(BB)
            ]
            pltpu.sync_copy(idx_hbm.at[pl.ds(c0, BB)], idx_b)
            db = [
                pltpu.async_copy(t2_hbm.at[idx_b.at[bb]], rows_b.at[bb], sem_b)
                for bb in range(BB)
            ]
            for d in da:
                d.wait()
            wa = pltpu.async_copy(rows_a, emb_hbm.at[pl.ds(a0, BB)], sem_wa)
            for d in db:
                d.wait()
            wb = pltpu.async_copy(rows_b, emb_hbm.at[pl.ds(c0, BB)], sem_wb)
            wa.wait()
            wb.wait()
            return carry

        lax.fori_loop(0, NPAIR, pair, 0)

    return gather_kernel(t2, idx2d)


def _p3_body(x_ref, idx_ref, eye_ref, o_ref):
    x = x_ref[...]
    hi = idx_ref[...] >= SPLIT
    eye = eye_ref[...]
    for f in range(FIELDS):
        xf = x[:, f, :]
        sel = jnp.where(hi[:, f][:, None], xf[:, TO_DIM:], xf[:, :TO_DIM])
        o_ref[f] = lax.dot_general(
            eye, sel, (((1,), (1,)), ((), ())),
            preferred_element_type=jnp.float32)


def _tc_finalize(emb2, idx2d, eye):
    return pl.pallas_call(
        _p3_body,
        grid=(BATCH // P3_BM,),
        in_specs=[
            pl.BlockSpec((P3_BM, FIELDS, 2 * TO_DIM), lambda i: (i, 0, 0)),
            pl.BlockSpec((P3_BM, FIELDS), lambda i: (i, 0)),
            pl.BlockSpec((TO_DIM, TO_DIM), lambda i: (0, 0)),
        ],
        out_specs=pl.BlockSpec((FIELDS, TO_DIM, P3_BM), lambda i: (0, 0, i)),
        out_shape=jax.ShapeDtypeStruct((FIELDS, TO_DIM, BATCH), jnp.float32),
    )(emb2, idx2d, eye)


def kernel(indexes, table, W, b):
    idx2d = indexes.astype(jnp.int32)
    k2d = jnp.where(idx2d >= SPLIT, idx2d - SPLIT, idx2d)
    t2 = _tc_transform(table.T, W.T, b.reshape(1, TO_DIM))
    emb2 = _sc_gather(t2, k2d)
    out_t = _tc_finalize(emb2, idx2d, jnp.eye(TO_DIM, dtype=jnp.float32))
    return jnp.transpose(out_t, (2, 0, 1))
